# baseline (speedup 1.0000x reference)
import functools

import jax
import jax.numpy as jnp
from jax import lax
from jax.experimental import pallas as pl
from jax.experimental.pallas import tpu as pltpu
from jax.experimental.pallas import tpu_sc as plsc

_NUM_TOKENS = 12
_BATCH = 16384
_LANES = 16
_NC, _NS = 2, 16
_NW = _NC * _NS
_CHUNK = _BATCH // _NW


@functools.partial(
    pl.kernel,
    out_type=jax.ShapeDtypeStruct((_BATCH,), jnp.float32),
    mesh=plsc.VectorSubcoreMesh(core_axis_name="c", subcore_axis_name="s"),
    compiler_params=pltpu.CompilerParams(needs_layout_passes=False),
    scratch_types=[
        pltpu.VMEM((_CHUNK,), jnp.int32),
        pltpu.VMEM((_CHUNK,), jnp.float32),
        pltpu.VMEM((_NUM_TOKENS,), jnp.float32),
        pltpu.VMEM((1,), jnp.float32),
        pltpu.SemaphoreType.DMA,
        pltpu.SemaphoreType.DMA,
        pltpu.SemaphoreType.DMA,
    ],
)
def _sc_lookup(idx_hbm, w_hbm, b_hbm, out_hbm, idx_v, out_v, w_v, b_v,
               sem_i, sem_w, sem_b):
    wid = lax.axis_index("s") * _NC + lax.axis_index("c")
    base = wid * _CHUNK

    cp_i = pltpu.async_copy(idx_hbm.at[pl.ds(base, _CHUNK)], idx_v, sem_i)
    cp_w = pltpu.async_copy(w_hbm, w_v, sem_w)
    cp_b = pltpu.async_copy(b_hbm, b_v, sem_b)
    cp_b.wait()
    b16 = plsc.load_gather(b_v, [jnp.zeros((_LANES,), jnp.int32)])
    cp_w.wait()
    cp_i.wait()

    _UNROLL = 4

    def body(i, carry):
        for u in range(_UNROLL):
            sl = pl.ds((i * _UNROLL + u) * _LANES, _LANES)
            idx = idx_v[sl] - 1
            out_v[sl] = jnp.maximum(plsc.load_gather(w_v, [idx]) + b16, 0.0)
        return carry

    lax.fori_loop(0, _CHUNK // (_LANES * _UNROLL), body, 0)

    pltpu.sync_copy(out_v, out_hbm.at[pl.ds(base, _CHUNK)])


def kernel(inputs, W, b):
    x = inputs.reshape(-1).astype(jnp.int32)
    out = _sc_lookup(x, W.reshape(-1), b)
    return out.reshape(_BATCH, 1)


# speedup vs baseline: 1.1032x; 1.1032x over previous
import functools

import jax
import jax.numpy as jnp
from jax import lax
from jax.experimental import pallas as pl
from jax.experimental.pallas import tpu as pltpu
from jax.experimental.pallas import tpu_sc as plsc

_NUM_TOKENS = 12
_BATCH = 16384
_LANES = 16
_NC, _NS = 1, 16
_NW = _NC * _NS
_CHUNK = _BATCH // _NW


@functools.partial(
    pl.kernel,
    out_type=jax.ShapeDtypeStruct((_BATCH,), jnp.float32),
    mesh=plsc.VectorSubcoreMesh(core_axis_name="c", subcore_axis_name="s", num_cores=1),
    compiler_params=pltpu.CompilerParams(needs_layout_passes=False),
    scratch_types=[
        pltpu.VMEM((_CHUNK,), jnp.int32),
        pltpu.VMEM((_CHUNK,), jnp.float32),
        pltpu.VMEM((_NUM_TOKENS,), jnp.float32),
        pltpu.VMEM((1,), jnp.float32),
        pltpu.SemaphoreType.DMA,
        pltpu.SemaphoreType.DMA,
        pltpu.SemaphoreType.DMA,
    ],
)
def _sc_lookup(idx_hbm, w_hbm, b_hbm, out_hbm, idx_v, out_v, w_v, b_v,
               sem_i, sem_w, sem_b):
    wid = lax.axis_index("s") * _NC + lax.axis_index("c")
    base = wid * _CHUNK

    cp_i = pltpu.async_copy(idx_hbm.at[pl.ds(base, _CHUNK)], idx_v, sem_i)
    cp_w = pltpu.async_copy(w_hbm, w_v, sem_w)
    cp_b = pltpu.async_copy(b_hbm, b_v, sem_b)
    cp_b.wait()
    b16 = plsc.load_gather(b_v, [jnp.zeros((_LANES,), jnp.int32)])
    cp_w.wait()
    cp_i.wait()

    _UNROLL = 4

    def body(i, carry):
        for u in range(_UNROLL):
            sl = pl.ds((i * _UNROLL + u) * _LANES, _LANES)
            idx = idx_v[sl] - 1
            out_v[sl] = jnp.maximum(plsc.load_gather(w_v, [idx]) + b16, 0.0)
        return carry

    lax.fori_loop(0, _CHUNK // (_LANES * _UNROLL), body, 0)

    pltpu.sync_copy(out_v, out_hbm.at[pl.ds(base, _CHUNK)])


def kernel(inputs, W, b):
    x = inputs.reshape(-1).astype(jnp.int32)
    out = _sc_lookup(x, W.reshape(-1), b)
    return out.reshape(_BATCH, 1)


# speedup vs baseline: 1.1137x; 1.0095x over previous
import functools

import jax
import jax.numpy as jnp
from jax import lax
from jax.experimental import pallas as pl
from jax.experimental.pallas import tpu as pltpu
from jax.experimental.pallas import tpu_sc as plsc

_NUM_TOKENS = 12
_BATCH = 16384
_LANES = 16
_NC, _NS = 1, 16
_NW = _NC * _NS
_CHUNK = _BATCH // _NW


@functools.partial(
    pl.kernel,
    out_type=jax.ShapeDtypeStruct((_BATCH,), jnp.float32),
    mesh=plsc.VectorSubcoreMesh(core_axis_name="c", subcore_axis_name="s", num_cores=1),
    compiler_params=pltpu.CompilerParams(needs_layout_passes=False),
    scratch_types=[
        pltpu.VMEM((_CHUNK,), jnp.int32),
        pltpu.VMEM((_CHUNK,), jnp.float32),
        pltpu.VMEM((_NUM_TOKENS,), jnp.float32),
        pltpu.VMEM((1,), jnp.float32),
        pltpu.SemaphoreType.DMA,
        pltpu.SemaphoreType.DMA,
        pltpu.SemaphoreType.DMA,
    ],
)
def _sc_lookup(idx_hbm, w_hbm, b_hbm, out_hbm, idx_v, out_v, w_v, b_v,
               sem_i, sem_w, sem_b):
    wid = lax.axis_index("s") * _NC + lax.axis_index("c")
    base = wid * _CHUNK

    cp_i = pltpu.async_copy(idx_hbm.at[pl.ds(base, _CHUNK)], idx_v, sem_i)
    cp_w = pltpu.async_copy(w_hbm, w_v, sem_w)
    cp_b = pltpu.async_copy(b_hbm, b_v, sem_b)
    cp_b.wait()
    b16 = plsc.load_gather(b_v, [jnp.zeros((_LANES,), jnp.int32)])
    cp_w.wait()
    cp_i.wait()

    @plsc.parallel_loop(0, _CHUNK // _LANES, unroll=4)
    def _(g):
        sl = pl.ds(g * _LANES, _LANES)
        idx = idx_v[sl] - 1
        out_v[sl] = jnp.maximum(plsc.load_gather(w_v, [idx]) + b16, 0.0)

    pltpu.sync_copy(out_v, out_hbm.at[pl.ds(base, _CHUNK)])


def kernel(inputs, W, b):
    x = inputs.reshape(-1).astype(jnp.int32)
    out = _sc_lookup(x, W.reshape(-1), b)
    return out.reshape(_BATCH, 1)
